# baseline (speedup 1.0000x reference)
import functools

import jax
import jax.numpy as jnp
from jax import lax
from jax.experimental import pallas as pl
from jax.experimental.pallas import tpu as pltpu
from jax.experimental.pallas import tpu_sc as plsc

FEAT = 1024

_info = plsc.get_sparse_core_info()
_NC, _NS = _info.num_cores, _info.num_subcores
_NW = _NC * _NS


@functools.partial(jax.jit, static_argnames=("length",))
def _sc_copy(table, length):
    rows_per_w = length // _NW
    mesh = plsc.VectorSubcoreMesh(core_axis_name="c", subcore_axis_name="s")

    @functools.partial(
        pl.kernel,
        mesh=mesh,
        out_type=jax.ShapeDtypeStruct((length, FEAT), table.dtype),
    )
    def body(table_hbm, out_hbm):
        wid = lax.axis_index("s") * _NC + lax.axis_index("c")
        base = wid * rows_per_w
        pltpu.sync_copy(
            table_hbm.at[pl.ds(base, rows_per_w)],
            out_hbm.at[pl.ds(base, rows_per_w)],
        )

    return body(table)


def kernel(x, table):
    return _sc_copy(table, x.shape[1])


# speedup vs baseline: 24.8388x; 24.8388x over previous
import functools

import jax
import jax.numpy as jnp
from jax import lax
from jax.experimental import pallas as pl
from jax.experimental.pallas import tpu as pltpu
from jax.experimental.pallas import tpu_sc as plsc

FEAT = 1024
CHUNK_ROWS = 32
NBUF = 3

_info = plsc.get_sparse_core_info()
_NC, _NS = _info.num_cores, _info.num_subcores
_NW = _NC * _NS


@functools.partial(jax.jit, static_argnames=("length",))
def _sc_copy(table, length):
    rows_per_w = length // _NW
    nch = rows_per_w // CHUNK_ROWS
    mesh = plsc.VectorSubcoreMesh(core_axis_name="c", subcore_axis_name="s")

    scratch = [pltpu.VMEM((NBUF, CHUNK_ROWS, FEAT), table.dtype)]
    scratch += [pltpu.SemaphoreType.DMA for _ in range(2 * NBUF)]

    @functools.partial(
        pl.kernel,
        mesh=mesh,
        out_type=jax.ShapeDtypeStruct((length, FEAT), table.dtype),
        scratch_types=scratch,
    )
    def body(table_hbm, out_hbm, bufs, *sems):
        in_sems, out_sems = sems[:NBUF], sems[NBUF:]
        wid = lax.axis_index("s") * _NC + lax.axis_index("c")
        base = wid * rows_per_w

        def start_in(g, b):
            return pltpu.async_copy(
                table_hbm.at[pl.ds(base + g * CHUNK_ROWS, CHUNK_ROWS)],
                bufs.at[b],
                in_sems[b],
            )

        def start_out(g, b):
            return pltpu.async_copy(
                bufs.at[b],
                out_hbm.at[pl.ds(base + g * CHUNK_ROWS, CHUNK_ROWS)],
                out_sems[b],
            )

        in_h = {}
        out_h = {}
        out_waited = set()
        for b in range(min(NBUF, nch)):
            in_h[b] = start_in(b, b)
        for g in range(nch):
            b = g % NBUF
            nxt = g + NBUF - 1
            if g >= 1 and nxt < nch:
                pb = (g - 1) % NBUF
                out_h[g - 1].wait()
                out_waited.add(g - 1)
                in_h[nxt] = start_in(nxt, pb)
            in_h[g].wait()
            out_h[g] = start_out(g, b)
        for g in range(nch):
            if g not in out_waited:
                out_h[g].wait()

    return body(table)


def kernel(x, table):
    return _sc_copy(table, x.shape[1])
